# baseline (speedup 1.0000x reference)
import functools

import jax
import jax.numpy as jnp
from jax import lax
from jax.experimental import pallas as pl
from jax.experimental.pallas import tpu as pltpu
from jax.experimental.pallas import tpu_sc as plsc

N = 49800
D_IN = 100
E = 796800

NC = 2
NS = 16
G = 128
NPAD = 49920
ROWS_T = NPAD // NS
E_PAD = 819200
ROWS_E = E_PAD // G
CI = 40

BR = 1992
NBLK = N // BR

_f32 = jnp.float32


def _mesh():
    return plsc.VectorSubcoreMesh(
        core_axis_name="c", subcore_axis_name="s", num_cores=NC, num_subcores=NS)


def _sc_degree(dst2d, zeros1):
    R = ROWS_E // (NS * NC)
    NSS = 8
    scratch = [
        pltpu.VMEM((R, G), jnp.int32),
        pltpu.VMEM((G,), _f32),
        pltpu.VMEM_SHARED((NPAD,), _f32),
    ] + [pltpu.SemaphoreType.DMA] * NSS

    def body(dst_h, z_h, out_h, dst_v, ones_v, shared, *sems):
        cid = lax.axis_index("c")
        sid = lax.axis_index("s")
        for i in range(G // 16):
            ones_v[pl.ds(16 * i, 16)] = jnp.ones((16,), _f32)
        pltpu.sync_copy(z_h, shared.at[pl.ds(sid * ROWS_T, ROWS_T)])
        plsc.subcore_barrier()
        row_base = (sid * NC + cid) * R
        pltpu.sync_copy(dst_h.at[pl.ds(row_base, R)], dst_v)
        for k in range(NSS):
            pltpu.async_copy(ones_v, shared.at[dst_v.at[k]], sems[k],
                             add=True)

        @pl.loop(0, R // NSS - 1)
        def _(i):
            for k in range(NSS):
                j = i * NSS + k
                pltpu.make_async_copy(
                    ones_v, shared.at[dst_v.at[j]], sems[k]).wait()
                pltpu.async_copy(
                    ones_v, shared.at[dst_v.at[j + NSS]], sems[k],
                    add=True)

        for k in range(NSS):
            pltpu.make_async_copy(
                ones_v, shared.at[dst_v.at[R - NSS + k]], sems[k]).wait()

        plsc.subcore_barrier()
        pltpu.sync_copy(shared.at[pl.ds(sid * ROWS_T, ROWS_T)],
                        out_h.at[pl.ds(cid * NPAD + sid * ROWS_T, ROWS_T)])

    fn = pl.kernel(body,
                   out_type=jax.ShapeDtypeStruct((NC * NPAD,), _f32),
                   mesh=_mesh(), scratch_types=scratch,
                   compiler_params=pltpu.CompilerParams(
                       use_tc_tiling_on_sc=False))
    return fn(dst2d, zeros1)


@functools.lru_cache(maxsize=None)
def _sc_scatter_kernel(d):
    R = ROWS_E // (NS * NC)
    RB = 5 if d > 16 else 10
    SL = 0 if d > 16 else 4
    nsem = RB + (RB if SL else 0)
    scratch = (
        [pltpu.VMEM((CI, G), jnp.int32)] * 2
        + [pltpu.VMEM((G, d), _f32) for _ in range(RB)]
        + [pltpu.VMEM_SHARED((NPAD, d), _f32)]
        + [pltpu.SemaphoreType.DMA] * nsem
    )

    def body(y_h, src_h, dst_h, z_h, out_h, src_v, dst_v, *rest):
        bufs = list(rest[:RB])
        shared = rest[RB]
        gsem = list(rest[RB + 1:2 * RB + 1])
        ssem = list(rest[2 * RB + 1:])
        cid = lax.axis_index("c")
        sid = lax.axis_index("s")

        pltpu.sync_copy(z_h, shared.at[pl.ds(sid * ROWS_T, ROWS_T)])
        plsc.subcore_barrier()

        def gwait(j, b):
            pltpu.make_async_copy(
                y_h.at[src_v.at[j]], bufs[b], gsem[b]).wait()

        def swait(j, b):
            pltpu.make_async_copy(
                bufs[b], shared.at[dst_v.at[j]], ssem[b]).wait()

        row_base = (sid * NC + cid) * R
        for ch in range(R // CI):
            pltpu.sync_copy(src_h.at[pl.ds(row_base + ch * CI, CI)], src_v)
            pltpu.sync_copy(dst_h.at[pl.ds(row_base + ch * CI, CI)], dst_v)

            if SL == 0:
                for b in range(RB):
                    pltpu.async_copy(y_h.at[src_v.at[b]], bufs[b], gsem[b])

                @pl.loop(0, (CI - RB) // RB)
                def _(i):
                    for b in range(RB):
                        j = i * RB + b
                        gwait(j, b)
                        pltpu.sync_copy(bufs[b], shared.at[dst_v.at[j]],
                                        add=True)
                        pltpu.async_copy(
                            y_h.at[src_v.at[j + RB]], bufs[b], gsem[b])

                for b in range(RB):
                    j = CI - RB + b
                    gwait(j, b)
                    pltpu.sync_copy(bufs[b], shared.at[dst_v.at[j]],
                                    add=True)
                continue

            for m in range(RB - SL):
                pltpu.async_copy(y_h.at[src_v.at[m]], bufs[m], gsem[m])
            for j in range(SL):
                gwait(j, j)
                pltpu.async_copy(bufs[j], shared.at[dst_v.at[j]],
                                 ssem[j], add=True)
                m = j + RB - SL
                pltpu.async_copy(y_h.at[src_v.at[m]], bufs[m], gsem[m])

            @pl.loop(0, (CI - RB) // RB)
            def _(i):
                for u in range(RB):
                    j = SL + i * RB + u
                    b = (SL + u) % RB
                    gwait(j, b)
                    pltpu.async_copy(bufs[b], shared.at[dst_v.at[j]],
                                     ssem[b], add=True)
                    swait(j - SL, u)
                    pltpu.async_copy(
                        y_h.at[src_v.at[j + RB - SL]], bufs[u], gsem[u])

            for t in range(RB - SL):
                j = CI - RB + SL + t
                b = j % RB
                gwait(j, b)
                pltpu.async_copy(bufs[b], shared.at[dst_v.at[j]],
                                 ssem[b], add=True)
                swait(j - SL, (j - SL) % RB)
            for t in range(SL):
                j = CI - SL + t
                swait(j, j % RB)

        plsc.subcore_barrier()
        pltpu.sync_copy(shared.at[pl.ds(sid * ROWS_T, ROWS_T)],
                        out_h.at[pl.ds(cid * NPAD + sid * ROWS_T, ROWS_T)])

    return pl.kernel(body,
                     out_type=jax.ShapeDtypeStruct((NC * NPAD, d), _f32),
                     mesh=_mesh(), scratch_types=scratch,
                     compiler_params=pltpu.CompilerParams(
                         use_tc_tiling_on_sc=False))


def _sc_scatter(y, src2d, dst2d, zeros2):
    d = y.shape[1]
    return _sc_scatter_kernel(d)(y, src2d, dst2d, zeros2)


@functools.lru_cache(maxsize=None)
def _sc_scatter2_kernel():
    d = 32
    R = ROWS_E // (NS * NC)
    RB = 5
    scratch = (
        [pltpu.VMEM((CI, G), jnp.int32)] * 2
        + [pltpu.VMEM((G, d), _f32) for _ in range(RB)]
        + [pltpu.VMEM_SHARED((NPAD, d), _f32)]
        + [pltpu.SemaphoreType.DMA] * RB
    )

    def body(ya_h, yb_h, src_h, dst_h, z_h, out_h, src_v, dst_v, *rest):
        bufs = list(rest[:RB])
        shared = rest[RB]
        gsem = list(rest[RB + 1:])
        cid = lax.axis_index("c")
        sid = lax.axis_index("s")
        row_base = (sid * NC + cid) * R

        def one_pass(y_h, out_base):
            pltpu.sync_copy(z_h, shared.at[pl.ds(sid * ROWS_T, ROWS_T)])
            plsc.subcore_barrier()
            for ch in range(R // CI):
                pltpu.sync_copy(src_h.at[pl.ds(row_base + ch * CI, CI)],
                                src_v)
                pltpu.sync_copy(dst_h.at[pl.ds(row_base + ch * CI, CI)],
                                dst_v)
                for b in range(RB):
                    pltpu.async_copy(y_h.at[src_v.at[b]], bufs[b], gsem[b])

                @pl.loop(0, (CI - RB) // RB)
                def _(i):
                    for b in range(RB):
                        j = i * RB + b
                        pltpu.make_async_copy(
                            y_h.at[src_v.at[j]], bufs[b], gsem[b]).wait()
                        pltpu.sync_copy(bufs[b], shared.at[dst_v.at[j]],
                                        add=True)
                        pltpu.async_copy(
                            y_h.at[src_v.at[j + RB]], bufs[b], gsem[b])

                for b in range(RB):
                    j = CI - RB + b
                    pltpu.make_async_copy(
                        y_h.at[src_v.at[j]], bufs[b], gsem[b]).wait()
                    pltpu.sync_copy(bufs[b], shared.at[dst_v.at[j]],
                                    add=True)

            plsc.subcore_barrier()
            pltpu.sync_copy(
                shared.at[pl.ds(sid * ROWS_T, ROWS_T)],
                out_h.at[pl.ds(out_base + cid * NPAD + sid * ROWS_T,
                               ROWS_T)])
            plsc.subcore_barrier()

        one_pass(ya_h, 0)
        one_pass(yb_h, NC * NPAD)

    return pl.kernel(body,
                     out_type=jax.ShapeDtypeStruct((2 * NC * NPAD, d), _f32),
                     mesh=_mesh(), scratch_types=scratch,
                     compiler_params=pltpu.CompilerParams(
                         use_tc_tiling_on_sc=False))


def _dinv_of(dT):
    return lax.rsqrt(dT[:, 0:1] + dT[:, 1:2] + 1.0)


def _full(shape):
    nd = len(shape)
    return pl.BlockSpec(shape, lambda i, _nd=nd: (0,) * nd)


def _tc_ffn_y1(xp, degT, W1f, c1, W2f, c2, Wg1):
    def body(x_r, dT_r, w1_r, c1_r, w2_r, c2_r, wg_r, ya_r, yb_r):
        dinv = _dinv_of(dT_r)
        h = jnp.maximum(
            jnp.dot(x_r[...], w1_r[...], preferred_element_type=_f32)
            + c1_r[...], 0.0)
        h2 = jnp.dot(h, w2_r[...], preferred_element_type=_f32) + c2_r[...]
        y = dinv * jnp.dot(h2, wg_r[...], preferred_element_type=_f32)
        ya_r[...] = y[:, :32]
        yb_r[...] = y[:, 32:]

    return pl.pallas_call(
        body, grid=(NBLK,),
        in_specs=[
            pl.BlockSpec((BR, D_IN), lambda i: (i, 0)),
            pl.BlockSpec((BR, 2), lambda i: (i, 0)),
            _full((D_IN, 400)), _full((1, 400)),
            _full((400, D_IN)), _full((1, D_IN)),
            _full((D_IN, 64)),
        ],
        out_specs=[pl.BlockSpec((BR, 32), lambda i: (i, 0))] * 2,
        out_shape=[jax.ShapeDtypeStruct((N, 32), _f32)] * 2,
    )(xp, degT, W1f, c1, W2f, c2, Wg1)


def _tc_layer(ss, ys, degT, b, W=None, slabs=(), final_cols=None):
    n_s = len(ss)
    dims = [y.shape[1] for y in ys]
    ss = [s.reshape(NC, NPAD, d) for s, d in zip(ss, dims)]
    db = b.shape[1]

    def body(*refs):
        s_rs = refs[:n_s]
        y_rs = refs[n_s:2 * n_s]
        dT_r = refs[2 * n_s]
        b_r = refs[2 * n_s + 1]
        rest = refs[2 * n_s + 2:]
        dinv = _dinv_of(dT_r)
        parts = [s_r[0] + s_r[1] + y_r[...] for s_r, y_r in zip(s_rs, y_rs)]
        m = parts[0] if n_s == 1 else jnp.concatenate(parts, axis=1)
        if final_cols is not None:
            o_r = rest[0]
            o_r[...] = jnp.maximum(
                dinv * m[:, :final_cols] + b_r[...], 0.0)
            return
        w_r = rest[0]
        o_rs = rest[1:]
        h = jnp.maximum(dinv * m[:, :db] + b_r[...], 0.0)
        y = dinv * jnp.dot(h, w_r[...], preferred_element_type=_f32)
        for o_r, (st, cov, emit) in zip(o_rs, slabs):
            sl = y[:, st:st + cov]
            if emit > cov:
                sl = jnp.concatenate(
                    [sl, jnp.zeros((BR, emit - cov), _f32)], axis=1)
            o_r[...] = sl

    in_specs = (
        [pl.BlockSpec((NC, BR, d), lambda i: (0, i, 0)) for d in dims]
        + [pl.BlockSpec((BR, d), lambda i: (i, 0)) for d in dims]
        + [pl.BlockSpec((BR, 2), lambda i: (i, 0)), _full((1, db))]
    )
    if final_cols is not None:
        out_specs = pl.BlockSpec((BR, final_cols), lambda i: (i, 0))
        out_shape = jax.ShapeDtypeStruct((N, final_cols), _f32)
        args = list(ss) + list(ys) + [degT, b]
    else:
        in_specs.append(_full(W.shape))
        out_specs = [pl.BlockSpec((BR, emit), lambda i: (i, 0))
                     for (_, _, emit) in slabs]
        out_shape = [jax.ShapeDtypeStruct((N, emit), _f32)
                     for (_, _, emit) in slabs]
        args = list(ss) + list(ys) + [degT, b, W]

    return pl.pallas_call(
        body, grid=(NBLK,), in_specs=in_specs,
        out_specs=out_specs, out_shape=out_shape,
    )(*args)


def _tc_head(hr, W_fc, b_fc):
    def body(h_r, w_r, b_r, o_r):
        o_r[...] = (jnp.dot(h_r[...], w_r[...], preferred_element_type=_f32)
                    + b_r[...])

    return pl.pallas_call(
        body,
        out_shape=jax.ShapeDtypeStruct((hr.shape[0], W_fc.shape[1]), _f32),
    )(hr, W_fc, b_fc)


def kernel(x, edge_index, W_ffn1, b_ffn1, bn1_g, bn1_b, bn1_m, bn1_v,
           W_ffn2, b_ffn2, bn2_g, bn2_b, bn2_m, bn2_v,
           W1, b1, W2, b2, W3, b3, W4, b4, W5, b5, W_fc, b_fc):
    r1 = lax.rsqrt(bn1_v + 1e-5) * bn1_g
    W1f = W_ffn1 * r1[None, :]
    c1 = ((b_ffn1 - bn1_m) * r1 + bn1_b)[None, :]
    r2 = lax.rsqrt(bn2_v + 1e-5) * bn2_g
    W2f = W_ffn2 * r2[None, :]
    c2 = ((b_ffn2 - bn2_m) * r2 + bn2_b)[None, :]

    epad = E_PAD - E
    src2d = jnp.concatenate(
        [edge_index[0], jnp.zeros((epad,), jnp.int32)]).reshape(ROWS_E, G)
    dst2d = jnp.concatenate(
        [edge_index[1], jnp.full((epad,), N, jnp.int32)]).reshape(ROWS_E, G)

    z1 = jnp.zeros((ROWS_T,), _f32)
    deg2 = _sc_degree(dst2d, z1)
    degT = deg2.reshape(NC, NPAD).T

    ya, yb = _tc_ffn_y1(x, degT, W1f, c1, W2f, c2, W1)

    z32 = jnp.zeros((ROWS_T, 32), _f32)
    z16 = jnp.zeros((ROWS_T, 16), _f32)
    z8 = jnp.zeros((ROWS_T, 8), _f32)

    s12 = _sc_scatter2_kernel()(ya, yb, src2d, dst2d, z32)
    s12 = s12.reshape(2, NC * NPAD, 32)
    (y2,) = _tc_layer((s12[0], s12[1]), (ya, yb), degT, b1.reshape(1, -1),
                      W2, slabs=[(0, 32, 32)])
    s2 = _sc_scatter(y2, src2d, dst2d, z32)
    (y3,) = _tc_layer((s2,), (y2,), degT, b2.reshape(1, -1),
                      W3, slabs=[(0, 16, 16)])
    s3 = _sc_scatter(y3, src2d, dst2d, z16)
    (y4,) = _tc_layer((s3,), (y3,), degT, b3.reshape(1, -1),
                      W4, slabs=[(0, 8, 8)])
    s4 = _sc_scatter(y4, src2d, dst2d, z8)
    (y5p,) = _tc_layer((s4,), (y4,), degT, b4.reshape(1, -1),
                       W5, slabs=[(0, 4, 8)])
    s5 = _sc_scatter(y5p, src2d, dst2d, z8)
    h5 = _tc_layer((s5,), (y5p,), degT, b5.reshape(1, -1),
                   final_cols=4)

    hr = h5.reshape(N * 4 // 1200, 1200)
    return _tc_head(hr, W_fc, b_fc.reshape(1, -1))


# speedup vs baseline: 1.1343x; 1.1343x over previous
import functools

import jax
import jax.numpy as jnp
from jax import lax
from jax.experimental import pallas as pl
from jax.experimental.pallas import tpu as pltpu
from jax.experimental.pallas import tpu_sc as plsc

N = 49800
D_IN = 100
E = 796800

NC = 2
NS = 16
G = 128
NPAD = 49920
ROWS_T = NPAD // NS
E_PAD = 819200
ROWS_E = E_PAD // G
CI = 40

BR = 1992
NBLK = N // BR

_f32 = jnp.float32


def _mesh():
    return plsc.VectorSubcoreMesh(
        core_axis_name="c", subcore_axis_name="s", num_cores=NC, num_subcores=NS)


def _sc_degree(dst2d, zeros1):
    R = ROWS_E // (NS * NC)
    NSS = 8
    scratch = [
        pltpu.VMEM((R, G), jnp.int32),
        pltpu.VMEM((G,), _f32),
        pltpu.VMEM_SHARED((NPAD,), _f32),
    ] + [pltpu.SemaphoreType.DMA] * NSS

    def body(dst_h, z_h, out_h, dst_v, ones_v, shared, *sems):
        cid = lax.axis_index("c")
        sid = lax.axis_index("s")
        for i in range(G // 16):
            ones_v[pl.ds(16 * i, 16)] = jnp.ones((16,), _f32)
        pltpu.sync_copy(z_h, shared.at[pl.ds(sid * ROWS_T, ROWS_T)])
        plsc.subcore_barrier()
        row_base = (sid * NC + cid) * R
        pltpu.sync_copy(dst_h.at[pl.ds(row_base, R)], dst_v)
        for k in range(NSS):
            pltpu.async_copy(ones_v, shared.at[dst_v.at[k]], sems[k],
                             add=True)

        @pl.loop(0, R // NSS - 1)
        def _(i):
            for k in range(NSS):
                j = i * NSS + k
                pltpu.make_async_copy(
                    ones_v, shared.at[dst_v.at[j]], sems[k]).wait()
                pltpu.async_copy(
                    ones_v, shared.at[dst_v.at[j + NSS]], sems[k],
                    add=True)

        for k in range(NSS):
            pltpu.make_async_copy(
                ones_v, shared.at[dst_v.at[R - NSS + k]], sems[k]).wait()

        plsc.subcore_barrier()
        pltpu.sync_copy(shared.at[pl.ds(sid * ROWS_T, ROWS_T)],
                        out_h.at[pl.ds(cid * NPAD + sid * ROWS_T, ROWS_T)])

    fn = pl.kernel(body,
                   out_type=jax.ShapeDtypeStruct((NC * NPAD,), _f32),
                   mesh=_mesh(), scratch_types=scratch,
                   compiler_params=pltpu.CompilerParams(
                       use_tc_tiling_on_sc=False))
    return fn(dst2d, zeros1)


@functools.lru_cache(maxsize=None)
def _sc_scatter_kernel(d):
    R = ROWS_E // (NS * NC)
    RB = 5 if d > 16 else 10
    SL = 0 if d > 16 else 4
    nsem = RB + (RB if SL else 0)
    scratch = (
        [pltpu.VMEM((CI, G), jnp.int32)] * 2
        + [pltpu.VMEM((G, d), _f32) for _ in range(RB)]
        + [pltpu.VMEM_SHARED((NPAD, d), _f32)]
        + [pltpu.SemaphoreType.DMA] * nsem
    )

    def body(y_h, src_h, dst_h, z_h, out_h, src_v, dst_v, *rest):
        bufs = list(rest[:RB])
        shared = rest[RB]
        gsem = list(rest[RB + 1:2 * RB + 1])
        ssem = list(rest[2 * RB + 1:])
        cid = lax.axis_index("c")
        sid = lax.axis_index("s")

        pltpu.sync_copy(z_h, shared.at[pl.ds(sid * ROWS_T, ROWS_T)])
        plsc.subcore_barrier()

        def gwait(j, b):
            pltpu.make_async_copy(
                y_h.at[src_v.at[j]], bufs[b], gsem[b]).wait()

        def swait(j, b):
            pltpu.make_async_copy(
                bufs[b], shared.at[dst_v.at[j]], ssem[b]).wait()

        row_base = (sid * NC + cid) * R
        for ch in range(R // CI):
            pltpu.sync_copy(src_h.at[pl.ds(row_base + ch * CI, CI)], src_v)
            pltpu.sync_copy(dst_h.at[pl.ds(row_base + ch * CI, CI)], dst_v)

            if SL == 0:
                for b in range(RB):
                    pltpu.async_copy(y_h.at[src_v.at[b]], bufs[b], gsem[b])

                @pl.loop(0, (CI - RB) // RB)
                def _(i):
                    for b in range(RB):
                        j = i * RB + b
                        gwait(j, b)
                        pltpu.sync_copy(bufs[b], shared.at[dst_v.at[j]],
                                        add=True)
                        pltpu.async_copy(
                            y_h.at[src_v.at[j + RB]], bufs[b], gsem[b])

                for b in range(RB):
                    j = CI - RB + b
                    gwait(j, b)
                    pltpu.sync_copy(bufs[b], shared.at[dst_v.at[j]],
                                    add=True)
                continue

            for m in range(RB - SL):
                pltpu.async_copy(y_h.at[src_v.at[m]], bufs[m], gsem[m])
            for j in range(SL):
                gwait(j, j)
                pltpu.async_copy(bufs[j], shared.at[dst_v.at[j]],
                                 ssem[j], add=True)
                m = j + RB - SL
                pltpu.async_copy(y_h.at[src_v.at[m]], bufs[m], gsem[m])

            @pl.loop(0, (CI - RB) // RB)
            def _(i):
                for u in range(RB):
                    j = SL + i * RB + u
                    b = (SL + u) % RB
                    gwait(j, b)
                    pltpu.async_copy(bufs[b], shared.at[dst_v.at[j]],
                                     ssem[b], add=True)
                    swait(j - SL, u)
                    pltpu.async_copy(
                        y_h.at[src_v.at[j + RB - SL]], bufs[u], gsem[u])

            for t in range(RB - SL):
                j = CI - RB + SL + t
                b = j % RB
                gwait(j, b)
                pltpu.async_copy(bufs[b], shared.at[dst_v.at[j]],
                                 ssem[b], add=True)
                swait(j - SL, (j - SL) % RB)
            for t in range(SL):
                j = CI - SL + t
                swait(j, j % RB)

        plsc.subcore_barrier()
        pltpu.sync_copy(shared.at[pl.ds(sid * ROWS_T, ROWS_T)],
                        out_h.at[pl.ds(cid * NPAD + sid * ROWS_T, ROWS_T)])

    return pl.kernel(body,
                     out_type=jax.ShapeDtypeStruct((NC * NPAD, d), _f32),
                     mesh=_mesh(), scratch_types=scratch,
                     compiler_params=pltpu.CompilerParams(
                         use_tc_tiling_on_sc=False))


def _sc_scatter(y, src2d, dst2d, zeros2):
    d = y.shape[1]
    return _sc_scatter_kernel(d)(y, src2d, dst2d, zeros2)


@functools.lru_cache(maxsize=None)
def _sc_scatter2_kernel():
    d = 32
    R = ROWS_E // (NS * NC)
    RB = 5
    scratch = (
        [pltpu.VMEM((CI, G), jnp.int32)] * 2
        + [pltpu.VMEM((G, d), _f32) for _ in range(RB)]
        + [pltpu.VMEM_SHARED((NPAD, d), _f32)]
        + [pltpu.SemaphoreType.DMA] * RB
    )

    def body(ya_h, yb_h, src_h, dst_h, z_h, out_h, src_v, dst_v, *rest):
        bufs = list(rest[:RB])
        shared = rest[RB]
        gsem = list(rest[RB + 1:])
        cid = lax.axis_index("c")
        sid = lax.axis_index("s")
        row_base = (sid * NC + cid) * R

        def one_pass(y_h, out_base):
            pltpu.sync_copy(z_h, shared.at[pl.ds(sid * ROWS_T, ROWS_T)])
            plsc.subcore_barrier()
            for ch in range(R // CI):
                pltpu.sync_copy(src_h.at[pl.ds(row_base + ch * CI, CI)],
                                src_v)
                pltpu.sync_copy(dst_h.at[pl.ds(row_base + ch * CI, CI)],
                                dst_v)
                for b in range(RB):
                    pltpu.async_copy(y_h.at[src_v.at[b]], bufs[b], gsem[b])

                @pl.loop(0, (CI - RB) // RB)
                def _(i):
                    for b in range(RB):
                        j = i * RB + b
                        pltpu.make_async_copy(
                            y_h.at[src_v.at[j]], bufs[b], gsem[b]).wait()
                        pltpu.sync_copy(bufs[b], shared.at[dst_v.at[j]],
                                        add=True)
                        pltpu.async_copy(
                            y_h.at[src_v.at[j + RB]], bufs[b], gsem[b])

                for b in range(RB):
                    j = CI - RB + b
                    pltpu.make_async_copy(
                        y_h.at[src_v.at[j]], bufs[b], gsem[b]).wait()
                    pltpu.sync_copy(bufs[b], shared.at[dst_v.at[j]],
                                    add=True)

            plsc.subcore_barrier()
            pltpu.sync_copy(
                shared.at[pl.ds(sid * ROWS_T, ROWS_T)],
                out_h.at[pl.ds(out_base + cid * NPAD + sid * ROWS_T,
                               ROWS_T)])
            plsc.subcore_barrier()

        one_pass(ya_h, 0)
        one_pass(yb_h, NC * NPAD)

    return pl.kernel(body,
                     out_type=jax.ShapeDtypeStruct((2 * NC * NPAD, d), _f32),
                     mesh=_mesh(), scratch_types=scratch,
                     compiler_params=pltpu.CompilerParams(
                         use_tc_tiling_on_sc=False))


def _dinv_of(dT):
    return lax.rsqrt(dT[:, 0:1] + dT[:, 1:2] + 1.0)


def _full(shape):
    nd = len(shape)
    return pl.BlockSpec(shape, lambda i, _nd=nd: (0,) * nd)


def _tc_ffn_y1(xp, degT, W1f, c1, W2f, c2, Wg1):
    def body(x_r, dT_r, w1_r, c1_r, w2_r, c2_r, wg_r, ya_r, yb_r):
        dinv = _dinv_of(dT_r)
        h = jnp.maximum(
            jnp.dot(x_r[...], w1_r[...], preferred_element_type=_f32)
            + c1_r[...], 0.0)
        h2 = jnp.dot(h, w2_r[...], preferred_element_type=_f32) + c2_r[...]
        y = dinv * jnp.dot(h2, wg_r[...], preferred_element_type=_f32)
        ya_r[...] = y[:, :32]
        yb_r[...] = y[:, 32:]

    return pl.pallas_call(
        body, grid=(NBLK,),
        in_specs=[
            pl.BlockSpec((BR, D_IN), lambda i: (i, 0)),
            pl.BlockSpec((BR, 2), lambda i: (i, 0)),
            _full((D_IN, 400)), _full((1, 400)),
            _full((400, D_IN)), _full((1, D_IN)),
            _full((D_IN, 64)),
        ],
        out_specs=[pl.BlockSpec((BR, 32), lambda i: (i, 0))] * 2,
        out_shape=[jax.ShapeDtypeStruct((N, 32), _f32)] * 2,
    )(xp, degT, W1f, c1, W2f, c2, Wg1)


def _tc_layer(ss, ys, degT, b, W=None, slabs=(), final_cols=None):
    n_s = len(ss)
    dims = [y.shape[1] for y in ys]
    ss = [s.reshape(NC, NPAD, d) for s, d in zip(ss, dims)]
    db = b.shape[1]

    def body(*refs):
        s_rs = refs[:n_s]
        y_rs = refs[n_s:2 * n_s]
        dT_r = refs[2 * n_s]
        b_r = refs[2 * n_s + 1]
        rest = refs[2 * n_s + 2:]
        dinv = _dinv_of(dT_r)
        parts = [s_r[0] + s_r[1] + y_r[...] for s_r, y_r in zip(s_rs, y_rs)]
        m = parts[0] if n_s == 1 else jnp.concatenate(parts, axis=1)
        if final_cols is not None:
            o_r = rest[0]
            o_r[...] = jnp.maximum(
                dinv * m[:, :final_cols] + b_r[...], 0.0)
            return
        w_r = rest[0]
        o_rs = rest[1:]
        h = jnp.maximum(dinv * m[:, :db] + b_r[...], 0.0)
        y = dinv * jnp.dot(h, w_r[...], preferred_element_type=_f32)
        for o_r, (st, cov, emit) in zip(o_rs, slabs):
            sl = y[:, st:st + cov]
            if emit > cov:
                sl = jnp.concatenate(
                    [sl, jnp.zeros((BR, emit - cov), _f32)], axis=1)
            o_r[...] = sl

    in_specs = (
        [pl.BlockSpec((NC, BR, d), lambda i: (0, i, 0)) for d in dims]
        + [pl.BlockSpec((BR, d), lambda i: (i, 0)) for d in dims]
        + [pl.BlockSpec((BR, 2), lambda i: (i, 0)), _full((1, db))]
    )
    if final_cols is not None:
        out_specs = pl.BlockSpec((BR, final_cols), lambda i: (i, 0))
        out_shape = jax.ShapeDtypeStruct((N, final_cols), _f32)
        args = list(ss) + list(ys) + [degT, b]
    else:
        in_specs.append(_full(W.shape))
        out_specs = [pl.BlockSpec((BR, emit), lambda i: (i, 0))
                     for (_, _, emit) in slabs]
        out_shape = [jax.ShapeDtypeStruct((N, emit), _f32)
                     for (_, _, emit) in slabs]
        args = list(ss) + list(ys) + [degT, b, W]

    return pl.pallas_call(
        body, grid=(NBLK,), in_specs=in_specs,
        out_specs=out_specs, out_shape=out_shape,
    )(*args)


def _tc_head(hr, W_fc, b_fc):
    def body(h_r, w_r, b_r, o_r):
        o_r[...] = (jnp.dot(h_r[...], w_r[...], preferred_element_type=_f32)
                    + b_r[...])

    return pl.pallas_call(
        body,
        out_shape=jax.ShapeDtypeStruct((hr.shape[0], W_fc.shape[1]), _f32),
    )(hr, W_fc, b_fc)


def kernel(x, edge_index, W_ffn1, b_ffn1, bn1_g, bn1_b, bn1_m, bn1_v,
           W_ffn2, b_ffn2, bn2_g, bn2_b, bn2_m, bn2_v,
           W1, b1, W2, b2, W3, b3, W4, b4, W5, b5, W_fc, b_fc):
    r1 = lax.rsqrt(bn1_v + 1e-5) * bn1_g
    W1f = W_ffn1 * r1[None, :]
    c1 = ((b_ffn1 - bn1_m) * r1 + bn1_b)[None, :]
    r2 = lax.rsqrt(bn2_v + 1e-5) * bn2_g
    W2f = W_ffn2 * r2[None, :]
    c2 = ((b_ffn2 - bn2_m) * r2 + bn2_b)[None, :]

    epad = E_PAD - E
    src2d = jnp.concatenate(
        [edge_index[0], jnp.zeros((epad,), jnp.int32)]).reshape(ROWS_E, G)
    dst2d = jnp.concatenate(
        [edge_index[1], jnp.full((epad,), N, jnp.int32)]).reshape(ROWS_E, G)

    z1 = jnp.zeros((ROWS_T,), _f32)
    deg2 = _sc_degree(dst2d, z1)
    degT = deg2.reshape(NC, NPAD).T

    ya, yb = _tc_ffn_y1(x, degT, W1f, c1, W2f, c2, W1)

    z32 = jnp.zeros((ROWS_T, 32), _f32)
    z16 = jnp.zeros((ROWS_T, 16), _f32)
    z8 = jnp.zeros((ROWS_T, 8), _f32)

    s1a = _sc_scatter(ya, src2d, dst2d, z32)
    s1b = _sc_scatter(yb, src2d, dst2d, z32)
    (y2,) = _tc_layer((s1a, s1b), (ya, yb), degT, b1.reshape(1, -1),
                      W2, slabs=[(0, 32, 32)])
    s2 = _sc_scatter(y2, src2d, dst2d, z32)
    (y3,) = _tc_layer((s2,), (y2,), degT, b2.reshape(1, -1),
                      W3, slabs=[(0, 16, 16)])
    s3 = _sc_scatter(y3, src2d, dst2d, z16)
    (y4,) = _tc_layer((s3,), (y3,), degT, b3.reshape(1, -1),
                      W4, slabs=[(0, 8, 8)])
    s4 = _sc_scatter(y4, src2d, dst2d, z8)
    (y5p,) = _tc_layer((s4,), (y4,), degT, b4.reshape(1, -1),
                       W5, slabs=[(0, 4, 8)])
    s5 = _sc_scatter(y5p, src2d, dst2d, z8)
    h5 = _tc_layer((s5,), (y5p,), degT, b5.reshape(1, -1),
                   final_cols=4)

    hr = h5.reshape(N * 4 // 1200, 1200)
    return _tc_head(hr, W_fc, b_fc.reshape(1, -1))
